# baseline (speedup 1.0000x reference)
import functools

import jax
import jax.numpy as jnp
from jax.experimental import pallas as pl
from jax.experimental.pallas import tpu as pltpu

NUM_EMB = 1024
DIM = 64
B = 8
HW = 1024
COMMIT = 0.25


def _vq_body(x_ref, e_ref, q_ref, idx_ref, loss_ref):
    x = x_ref[0]
    e = e_ref[...]
    enorm = jnp.sum(e * e, axis=1, keepdims=True)
    xnorm = jnp.sum(x * x, axis=0, keepdims=True)
    mm = jax.lax.dot_general(e, x, (((1,), (0,)), ((), ())),
                             preferred_element_type=jnp.float32)
    d = (xnorm + enorm) - 2.0 * mm
    vmin = jnp.min(d, axis=0, keepdims=True)
    iota = jax.lax.broadcasted_iota(jnp.int32, (NUM_EMB, HW), 0)
    idx = jnp.min(jnp.where(d == vmin, iota, NUM_EMB), axis=0, keepdims=True)
    idx_ref[0] = idx.astype(jnp.int32)
    onehot = (iota == idx).astype(jnp.float32)
    q_ref[0] = jax.lax.dot_general(e, onehot, (((0,), (0,)), ((), ())),
                                   preferred_element_type=jnp.float32)
    loss_ref[...] = jnp.sum(vmin, keepdims=True).reshape(1, 1, 1)


@functools.partial(jax.jit, static_argnames=())
def kernel(inputs, embedding_weight):
    x = inputs.reshape(B, DIM, HW)
    q, idx, loss = pl.pallas_call(
        _vq_body,
        grid=(B,),
        in_specs=[
            pl.BlockSpec((1, DIM, HW), lambda b: (b, 0, 0)),
            pl.BlockSpec((NUM_EMB, DIM), lambda b: (0, 0)),
        ],
        out_specs=[
            pl.BlockSpec((1, DIM, HW), lambda b: (b, 0, 0)),
            pl.BlockSpec((1, 1, HW), lambda b: (b, 0, 0)),
            pl.BlockSpec((1, 1, 1), lambda b: (b, 0, 0)),
        ],
        out_shape=[
            jax.ShapeDtypeStruct((B, DIM, HW), jnp.float32),
            jax.ShapeDtypeStruct((B, 1, HW), jnp.int32),
            jax.ShapeDtypeStruct((B, 1, 1), jnp.float32),
        ],
        compiler_params=pltpu.CompilerParams(
            dimension_semantics=("parallel",)),
    )(x, embedding_weight)
    quantized_st = q.reshape(inputs.shape)
    vq_loss = jnp.sum(loss) * ((1.0 + COMMIT) / (B * HW * DIM))
    indices = idx.reshape(B, 32, 32)
    return quantized_st, vq_loss, indices


# speedup vs baseline: 1.0474x; 1.0474x over previous
import functools

import jax
import jax.numpy as jnp
from jax.experimental import pallas as pl
from jax.experimental.pallas import tpu as pltpu

NUM_EMB = 1024
DIM = 64
B = 8
HW = 1024
COMMIT = 0.25


def _vq_body(x_ref, e_ref, q_ref, idx_ref, loss_ref):
    x = x_ref[0]
    e = e_ref[...]
    enorm = jnp.sum(e * e, axis=1, keepdims=True)
    xnorm = jnp.sum(x * x, axis=0, keepdims=True)
    mm2 = jax.lax.dot_general(e + e, x, (((1,), (0,)), ((), ())),
                              preferred_element_type=jnp.float32)
    d = (xnorm + enorm) - mm2
    vmin = jnp.min(d, axis=0, keepdims=True)
    iota = jax.lax.broadcasted_iota(jnp.int32, (NUM_EMB, HW), 0)
    idx = jnp.min(jnp.where(d == vmin, iota, NUM_EMB), axis=0, keepdims=True)
    idx = idx.astype(jnp.int32)
    idx_ref[0] = idx
    onehot = (iota == idx).astype(jnp.float32)
    q = jax.lax.dot_general(e, onehot, (((0,), (0,)), ((), ())),
                            preferred_element_type=jnp.float32)
    q_ref[0] = q
    b = pl.program_id(0)
    r = q - x
    part = jnp.sum(r * r, keepdims=True).reshape(1, 1)

    @pl.when(b == 0)
    def _():
        loss_ref[...] = jnp.zeros((1, 1), jnp.float32)

    acc = loss_ref[...] + part
    loss_ref[...] = jnp.where(b == B - 1,
                              acc * ((1.0 + COMMIT) / (B * HW * DIM)), acc)


@functools.partial(jax.jit, static_argnames=())
def kernel(inputs, embedding_weight):
    x = inputs.reshape(B, DIM, HW)
    q, idx, loss = pl.pallas_call(
        _vq_body,
        grid=(B,),
        in_specs=[
            pl.BlockSpec((1, DIM, HW), lambda b: (b, 0, 0)),
            pl.BlockSpec((NUM_EMB, DIM), lambda b: (0, 0)),
        ],
        out_specs=[
            pl.BlockSpec((1, DIM, HW), lambda b: (b, 0, 0)),
            pl.BlockSpec((1, 1, HW), lambda b: (b, 0, 0)),
            pl.BlockSpec((1, 1), lambda b: (0, 0)),
        ],
        out_shape=[
            jax.ShapeDtypeStruct((B, DIM, HW), jnp.float32),
            jax.ShapeDtypeStruct((B, 1, HW), jnp.int32),
            jax.ShapeDtypeStruct((1, 1), jnp.float32),
        ],
    )(x, embedding_weight)
    quantized_st = q.reshape(inputs.shape)
    vq_loss = loss[0, 0]
    indices = idx.reshape(B, 32, 32)
    return quantized_st, vq_loss, indices
